# speedup vs baseline: 8.4945x; 8.4945x over previous
import jax
import jax.numpy as jnp
from jax import lax
from jax.experimental import pallas as pl
from jax.experimental.pallas import tpu as pltpu
from jax.experimental.pallas import tpu_sc as plsc

D = 128

NC = 2
NS = 16
NW = NC * NS

B = 4096 * 200
PER_W = B // NW
CHUNK = 128
NCHUNK = PER_W // CHUNK
NBUF = 4


def _gather_body(x_hbm, table_hbm, out_hbm, idx_v, rows_v, *sems):
    gsems = sems[:NBUF]
    wsems = sems[NBUF:]
    wid = lax.axis_index("s") * NC + lax.axis_index("c")
    row0 = wid * NCHUNK

    pltpu.sync_copy(x_hbm.at[pl.ds(row0, NCHUNK)], idx_v)

    def out_slice(j):
        return out_hbm.at[pl.ds((row0 + j) * CHUNK, CHUNK)]

    for b in range(NBUF):
        pltpu.async_copy(table_hbm.at[idx_v.at[b]], rows_v.at[b], gsems[b])

    def step(i, carry):
        g0 = i * NBUF
        for b in range(NBUF):
            j = g0 + b
            pltpu.make_async_copy(
                table_hbm.at[idx_v.at[j]], rows_v.at[b], gsems[b]
            ).wait()
            pltpu.async_copy(rows_v.at[b], out_slice(j), wsems[b])
            pltpu.make_async_copy(rows_v.at[b], out_slice(j), wsems[b]).wait()
            pltpu.async_copy(
                table_hbm.at[idx_v.at[j + NBUF]], rows_v.at[b], gsems[b]
            )
        return carry

    n_steady = NCHUNK // NBUF - 1
    lax.fori_loop(0, n_steady, step, 0)

    for b in range(NBUF):
        j = NCHUNK - NBUF + b
        pltpu.make_async_copy(
            table_hbm.at[idx_v.at[j]], rows_v.at[b], gsems[b]
        ).wait()
        pltpu.async_copy(rows_v.at[b], out_slice(j), wsems[b])
    for b in range(NBUF):
        j = NCHUNK - NBUF + b
        pltpu.make_async_copy(rows_v.at[b], out_slice(j), wsems[b]).wait()


_gather = pl.kernel(
    _gather_body,
    out_type=jax.ShapeDtypeStruct((B, D), jnp.float32),
    mesh=plsc.VectorSubcoreMesh(core_axis_name="c", subcore_axis_name="s"),
    scratch_types=(
        [
            pltpu.VMEM((NCHUNK, CHUNK), jnp.int32),
            pltpu.VMEM((NBUF, CHUNK, D), jnp.float32),
        ]
        + [pltpu.SemaphoreType.DMA] * (2 * NBUF)
    ),
)


def kernel(x, embedding, new_embedding):
    table = jnp.concatenate([embedding, new_embedding], axis=0)
    x2d = x.reshape(B // CHUNK, CHUNK).astype(jnp.int32)
    out = _gather(x2d, table)
    return out.reshape(x.shape[0], x.shape[1], D)


# speedup vs baseline: 8.5340x; 1.0047x over previous
import jax
import jax.numpy as jnp
from jax import lax
from jax.experimental import pallas as pl
from jax.experimental.pallas import tpu as pltpu
from jax.experimental.pallas import tpu_sc as plsc

D = 128

NC = 2
NS = 16
NW = NC * NS

B = 4096 * 200
PER_W = B // NW
CHUNK = 128
NCHUNK = PER_W // CHUNK
NBUF = 5


def _gather_body(x_hbm, table_hbm, out_hbm, idx_v, rows_v, *sems):
    gsems = sems[:NBUF]
    wsems = sems[NBUF:]
    wid = lax.axis_index("s") * NC + lax.axis_index("c")
    row0 = wid * NCHUNK

    pltpu.sync_copy(x_hbm.at[pl.ds(row0, NCHUNK)], idx_v)

    def out_slice(j):
        return out_hbm.at[pl.ds((row0 + j) * CHUNK, CHUNK)]

    for b in range(NBUF):
        pltpu.async_copy(table_hbm.at[idx_v.at[b]], rows_v.at[b], gsems[b])

    def step(i, carry):
        g0 = i * NBUF
        for b in range(NBUF):
            j = g0 + b
            pltpu.make_async_copy(
                table_hbm.at[idx_v.at[j]], rows_v.at[b], gsems[b]
            ).wait()
            pltpu.async_copy(rows_v.at[b], out_slice(j), wsems[b])
            pltpu.make_async_copy(rows_v.at[b], out_slice(j), wsems[b]).wait()
            pltpu.async_copy(
                table_hbm.at[idx_v.at[j + NBUF]], rows_v.at[b], gsems[b]
            )
        return carry

    n_steady = NCHUNK // NBUF - 1
    lax.fori_loop(0, n_steady, step, 0)

    for b in range(NBUF):
        j = NCHUNK - NBUF + b
        pltpu.make_async_copy(
            table_hbm.at[idx_v.at[j]], rows_v.at[b], gsems[b]
        ).wait()
        pltpu.async_copy(rows_v.at[b], out_slice(j), wsems[b])
    for b in range(NBUF):
        j = NCHUNK - NBUF + b
        pltpu.make_async_copy(rows_v.at[b], out_slice(j), wsems[b]).wait()


_gather = pl.kernel(
    _gather_body,
    out_type=jax.ShapeDtypeStruct((B, D), jnp.float32),
    mesh=plsc.VectorSubcoreMesh(core_axis_name="c", subcore_axis_name="s"),
    scratch_types=(
        [
            pltpu.VMEM((NCHUNK, CHUNK), jnp.int32),
            pltpu.VMEM((NBUF, CHUNK, D), jnp.float32),
        ]
        + [pltpu.SemaphoreType.DMA] * (2 * NBUF)
    ),
)


def kernel(x, embedding, new_embedding):
    table = jnp.concatenate([embedding, new_embedding], axis=0)
    x2d = x.reshape(B // CHUNK, CHUNK).astype(jnp.int32)
    out = _gather(x2d, table)
    return out.reshape(x.shape[0], x.shape[1], D)
